# baseline (speedup 1.0000x reference)
import functools

import jax
import jax.numpy as jnp
from jax import lax
from jax.experimental import pallas as pl
from jax.experimental.pallas import tpu as pltpu
from jax.experimental.pallas import tpu_sc as plsc

N_BINS = 15
ACC = 1792
NW = 32
ROWS_PER_CHUNK = 400
COL_PERIOD = 25


def _sc_body(n_chunks, sm_hbm, lab_hbm, btab_hbm, ctab_hbm, out_hbm,
             buf0, buf1, lbuf0, lbuf1, btab_v, ctab_v, conf_v, lacc_v,
             s0, s1, sl0, sl1):
    cid = lax.axis_index("c")
    sid = lax.axis_index("s")
    wid = sid * 2 + cid

    pltpu.sync_copy(btab_hbm, btab_v)
    pltpu.sync_copy(ctab_hbm, ctab_v)

    zf = jnp.zeros((16,), jnp.float32)

    def zero_conf(i, carry):
        conf_v[pl.ds(i * 16, 16)] = zf
        return carry

    lax.fori_loop(0, ACC // 16, zero_conf, 0)

    def zero_lacc(i, carry):
        lacc_v[pl.ds(i * 16, 16)] = zf
        return carry

    lax.fori_loop(0, ACC, zero_lacc, 0)

    iota = lax.iota(jnp.int32, 16)
    iota100 = iota * 100
    iota16 = iota * 16
    ones = jnp.ones((16,), jnp.float32)
    cols = [ctab_v[pl.ds(vi * 16, 16)] for vi in range(COL_PERIOD)]

    bufs = (buf0, buf1)
    lbufs = (lbuf0, lbuf1)
    sems = (s0, s1)
    lsems = (sl0, sl1)

    def start(c, b):
        off = pl.multiple_of(c * 40000, 8)
        loff = pl.multiple_of(c * 400, 8)
        pltpu.async_copy(sm_hbm.at[pl.ds(off, 40000)], bufs[b], sems[b])
        pltpu.async_copy(lab_hbm.at[pl.ds(loff, 400)], lbufs[b], lsems[b])

    def wait(b):
        pltpu.make_async_copy(sm_hbm.at[pl.ds(0, 40000)], bufs[b], sems[b]).wait()
        pltpu.make_async_copy(lab_hbm.at[pl.ds(0, 400)], lbufs[b], lsems[b]).wait()

    def bin_of(x):
        k = (x * 15.0 + 0.5).astype(jnp.int32)
        u = plsc.load_gather(btab_v, [k])
        return k - (x <= u).astype(jnp.int32)

    def process(buf, lbuf):
        def chunk_body(vo, carry):
            base = vo * (COL_PERIOD * 16)
            for vi in range(COL_PERIOD):
                x = buf[pl.ds(base + vi * 16, 16)]
                b = bin_of(x)
                idx = b * 100 + cols[vi]
                plsc.addupdate_scatter(conf_v, [idx], x)
            return carry

        lax.fori_loop(0, ROWS_PER_CHUNK * 100 // (COL_PERIOD * 16),
                      chunk_body, 0)

        for li in range(ROWS_PER_CHUNK // 16):
            lv = lbuf[pl.ds(li * 16, 16)]
            flat = iota100 + (li * 1600 + lv)
            xg = plsc.load_gather(buf, [flat])
            b = bin_of(xg)
            slot = (b * 100 + lv + 100) * 16 + iota
            plsc.addupdate_scatter(lacc_v, [slot], ones)

    @pl.when(wid < n_chunks)
    def _():
        start(wid, 0)

    @pl.when(wid + NW < n_chunks)
    def _():
        start(wid + NW, 1)

    n_outer = (n_chunks + 2 * NW - 1) // (2 * NW)

    def outer(i, carry):
        for b2 in range(2):
            c = wid + (2 * i + b2) * NW

            @pl.when(c < n_chunks)
            def _():
                wait(b2)
                process(bufs[b2], lbufs[b2])
                cn = c + 2 * NW

                @pl.when(cn < n_chunks)
                def _():
                    start(cn, b2)

        return carry

    lax.fori_loop(0, n_outer, outer, 0)

    def fold_body(ov, carry):
        s = conf_v[pl.ds(ov * 16, 16)]
        gbase = iota16 + ov * 256
        for l in range(16):
            s = s - plsc.load_gather(lacc_v, [gbase + l])
        conf_v[pl.ds(ov * 16, 16)] = s
        return carry

    lax.fori_loop(0, ACC // 16, fold_body, 0)

    pltpu.sync_copy(conf_v, out_hbm.at[wid])


def _final_body(inv_ncl, p_ref, o_ref):
    p = p_ref[...]
    s = jnp.sum(p, axis=0, keepdims=True)
    j = lax.broadcasted_iota(jnp.int32, (1, ACC), 1)
    m = (j >= 100) & (j < 1600)
    d = jnp.where(m, jnp.abs(s), 0.0)
    o_ref[...] = (jnp.sum(d) * inv_ncl).reshape(1, 1)


@jax.jit
def kernel(softmaxes, labels):
    n, num_classes = softmaxes.shape
    assert num_classes == 100 and n % ROWS_PER_CHUNK == 0
    n_chunks = n // ROWS_PER_CHUNK

    sm_flat = softmaxes.reshape(-1)
    lab = labels.astype(jnp.int32)
    btab = jnp.linspace(0.0, 1.0, N_BINS + 1).astype(jnp.float32)
    coltab = (jnp.arange(COL_PERIOD * 16, dtype=jnp.int32) % 100) + 100

    mesh = plsc.VectorSubcoreMesh(core_axis_name="c", subcore_axis_name="s")
    sc = pl.kernel(
        functools.partial(_sc_body, n_chunks),
        out_type=jax.ShapeDtypeStruct((NW, ACC), jnp.float32),
        mesh=mesh,
        compiler_params=pltpu.CompilerParams(needs_layout_passes=False),
        scratch_types=[
            pltpu.VMEM((40000,), jnp.float32),
            pltpu.VMEM((40000,), jnp.float32),
            pltpu.VMEM((400,), jnp.int32),
            pltpu.VMEM((400,), jnp.int32),
            pltpu.VMEM((16,), jnp.float32),
            pltpu.VMEM((COL_PERIOD * 16,), jnp.int32),
            pltpu.VMEM((ACC,), jnp.float32),
            pltpu.VMEM((ACC * 16,), jnp.float32),
            pltpu.SemaphoreType.DMA,
            pltpu.SemaphoreType.DMA,
            pltpu.SemaphoreType.DMA,
            pltpu.SemaphoreType.DMA,
        ],
    )
    part = sc(sm_flat, lab, btab, coltab)

    inv_ncl = 1.0 / (float(n) * float(num_classes))
    out = pl.pallas_call(
        functools.partial(_final_body, inv_ncl),
        out_shape=jax.ShapeDtypeStruct((1, 1), jnp.float32),
    )(part)
    return out[0, 0]


# speedup vs baseline: 1.9744x; 1.9744x over previous
import functools

import jax
import jax.numpy as jnp
from jax import lax
from jax.experimental import pallas as pl
from jax.experimental.pallas import tpu as pltpu
from jax.experimental.pallas import tpu_sc as plsc

N_BINS = 15
ACC = 1792
NW = 32
ROWS_PER_CHUNK = 400
COL_PERIOD = 25


def _sc_body(n_chunks, sm_hbm, lab_hbm, btab_hbm, ctab_hbm, out_hbm,
             buf0, buf1, lbuf0, lbuf1, btab_v, ctab_v, conf_v, lacc_v,
             s0, s1, sl0, sl1):
    cid = lax.axis_index("c")
    sid = lax.axis_index("s")
    wid = sid * 2 + cid

    pltpu.sync_copy(btab_hbm, btab_v)
    pltpu.sync_copy(ctab_hbm, ctab_v)

    zf = jnp.zeros((16,), jnp.float32)

    @plsc.parallel_loop(0, ACC // 16, unroll=8)
    def _(i):
        conf_v[pl.ds(i * 16, 16)] = zf

    @plsc.parallel_loop(0, ACC, unroll=8)
    def _(i):
        lacc_v[pl.ds(i * 16, 16)] = zf

    iota = lax.iota(jnp.int32, 16)
    iota100 = iota * 100
    iota16 = iota * 16
    ones = jnp.ones((16,), jnp.float32)
    cols = [ctab_v[pl.ds(vi * 16, 16)] for vi in range(COL_PERIOD)]

    bufs = (buf0, buf1)
    lbufs = (lbuf0, lbuf1)
    sems = (s0, s1)
    lsems = (sl0, sl1)

    def start(c, b):
        off = pl.multiple_of(c * 40000, 8)
        loff = pl.multiple_of(c * 400, 8)
        pltpu.async_copy(sm_hbm.at[pl.ds(off, 40000)], bufs[b], sems[b])
        pltpu.async_copy(lab_hbm.at[pl.ds(loff, 400)], lbufs[b], lsems[b])

    def wait(b):
        pltpu.make_async_copy(sm_hbm.at[pl.ds(0, 40000)], bufs[b], sems[b]).wait()
        pltpu.make_async_copy(lab_hbm.at[pl.ds(0, 400)], lbufs[b], lsems[b]).wait()

    def bin_of(x):
        k = (x * 15.0 + 0.5).astype(jnp.int32)
        u = plsc.load_gather(btab_v, [(k << 4) + iota])
        return k - (x <= u).astype(jnp.int32)

    def process(buf, lbuf):
        @plsc.parallel_loop(0, ROWS_PER_CHUNK * 100 // (COL_PERIOD * 16))
        def _(vo):
            base = vo * (COL_PERIOD * 16)
            for vi in range(COL_PERIOD):
                x = buf[pl.ds(base + vi * 16, 16)]
                b = bin_of(x)
                idx = b * 100 + cols[vi]
                plsc.addupdate_scatter(conf_v, [idx], x)

        @plsc.parallel_loop(0, ROWS_PER_CHUNK // 16, unroll=2)
        def _(li):
            lv = lbuf[pl.ds(li * 16, 16)]
            flat = iota100 + (li * 1600 + lv)
            xg = plsc.load_gather(buf, [flat])
            b = bin_of(xg)
            slot = (b * 100 + lv + 100) * 16 + iota
            plsc.addupdate_scatter(lacc_v, [slot], ones)

    @pl.when(wid < n_chunks)
    def _():
        start(wid, 0)

    @pl.when(wid + NW < n_chunks)
    def _():
        start(wid + NW, 1)

    n_outer = (n_chunks + 2 * NW - 1) // (2 * NW)

    def outer(i, carry):
        for b2 in range(2):
            c = wid + (2 * i + b2) * NW

            @pl.when(c < n_chunks)
            def _():
                wait(b2)
                process(bufs[b2], lbufs[b2])
                cn = c + 2 * NW

                @pl.when(cn < n_chunks)
                def _():
                    start(cn, b2)

        return carry

    lax.fori_loop(0, n_outer, outer, 0)

    @plsc.parallel_loop(0, ACC // 16)
    def _(ov):
        s = conf_v[pl.ds(ov * 16, 16)]
        gbase = iota16 + ov * 256
        for l in range(16):
            s = s - plsc.load_gather(lacc_v, [gbase + l])
        conf_v[pl.ds(ov * 16, 16)] = s

    pltpu.sync_copy(conf_v, out_hbm.at[wid])


def _final_body(inv_ncl, p_ref, o_ref):
    p = p_ref[...]
    s = jnp.sum(p, axis=0, keepdims=True)
    j = lax.broadcasted_iota(jnp.int32, (1, ACC), 1)
    m = (j >= 100) & (j < 1600)
    d = jnp.where(m, jnp.abs(s), 0.0)
    o_ref[...] = (jnp.sum(d) * inv_ncl).reshape(1, 1)


@jax.jit
def kernel(softmaxes, labels):
    n, num_classes = softmaxes.shape
    assert num_classes == 100 and n % ROWS_PER_CHUNK == 0
    n_chunks = n // ROWS_PER_CHUNK

    sm_flat = softmaxes.reshape(-1)
    lab = labels.astype(jnp.int32)
    btab = jnp.repeat(jnp.linspace(0.0, 1.0, N_BINS + 1).astype(jnp.float32), 16)
    coltab = (jnp.arange(COL_PERIOD * 16, dtype=jnp.int32) % 100) + 100

    mesh = plsc.VectorSubcoreMesh(core_axis_name="c", subcore_axis_name="s")
    sc = pl.kernel(
        functools.partial(_sc_body, n_chunks),
        out_type=jax.ShapeDtypeStruct((NW, ACC), jnp.float32),
        mesh=mesh,
        compiler_params=pltpu.CompilerParams(needs_layout_passes=False),
        scratch_types=[
            pltpu.VMEM((40000,), jnp.float32),
            pltpu.VMEM((40000,), jnp.float32),
            pltpu.VMEM((400,), jnp.int32),
            pltpu.VMEM((400,), jnp.int32),
            pltpu.VMEM((256,), jnp.float32),
            pltpu.VMEM((COL_PERIOD * 16,), jnp.int32),
            pltpu.VMEM((ACC,), jnp.float32),
            pltpu.VMEM((ACC * 16,), jnp.float32),
            pltpu.SemaphoreType.DMA,
            pltpu.SemaphoreType.DMA,
            pltpu.SemaphoreType.DMA,
            pltpu.SemaphoreType.DMA,
        ],
    )
    part = sc(sm_flat, lab, btab, coltab)

    inv_ncl = 1.0 / (float(n) * float(num_classes))
    out = pl.pallas_call(
        functools.partial(_final_body, inv_ncl),
        out_shape=jax.ShapeDtypeStruct((1, 1), jnp.float32),
    )(part)
    return out[0, 0]


# speedup vs baseline: 3.2841x; 1.6634x over previous
import functools

import jax
import jax.numpy as jnp
from jax import lax
from jax.experimental import pallas as pl
from jax.experimental.pallas import tpu as pltpu
from jax.experimental.pallas import tpu_sc as plsc

N_BINS = 15
ACC = 1792
NW = 32
ROWS_PER_CHUNK = 160
COL_PERIOD = 25


def _sc_body(n_chunks, sm_hbm, lab_hbm, btab_hbm, rtab_hbm, ctab_hbm, out_hbm,
             buf0, buf1, lbuf0, lbuf1, btab_v, rtab_v, ctab_v, conf_v, lacc_v,
             s0, s1, sl0, sl1):
    cid = lax.axis_index("c")
    sid = lax.axis_index("s")
    wid = sid * 2 + cid

    pltpu.sync_copy(btab_hbm, btab_v)
    pltpu.sync_copy(rtab_hbm, rtab_v)
    pltpu.sync_copy(ctab_hbm, ctab_v)

    zf = jnp.zeros((16,), jnp.float32)

    @plsc.parallel_loop(0, ACC // 16, unroll=8)
    def _(i):
        conf_v[pl.ds(i * 16, 16)] = zf

    @plsc.parallel_loop(0, ACC, unroll=8)
    def _(i):
        lacc_v[pl.ds(i * 16, 16)] = zf

    iota = lax.iota(jnp.int32, 16)
    iota16 = iota * 16
    zeros_i = jnp.zeros((16,), jnp.int32)
    ones = jnp.ones((16,), jnp.float32)

    bufs = (buf0, buf1)
    lbufs = (lbuf0, lbuf1)
    sems = (s0, s1)
    lsems = (sl0, sl1)

    def start(c, b):
        off = pl.multiple_of(c * ROWS_PER_CHUNK, 8)
        pltpu.async_copy(sm_hbm.at[pl.ds(off, ROWS_PER_CHUNK)], bufs[b], sems[b])
        pltpu.async_copy(lab_hbm.at[pl.ds(off, ROWS_PER_CHUNK)], lbufs[b], lsems[b])

    def wait(b):
        pltpu.make_async_copy(
            sm_hbm.at[pl.ds(0, ROWS_PER_CHUNK)], bufs[b], sems[b]).wait()
        pltpu.make_async_copy(
            lab_hbm.at[pl.ds(0, ROWS_PER_CHUNK)], lbufs[b], lsems[b]).wait()

    def bin_adj(x):
        k = (x * 15.0 + 0.5).astype(jnp.int32)
        u = plsc.load_gather(btab_v, [(k << 4) + iota])
        sel = jnp.where(x <= u, 0, 1)
        return k + sel

    def process(buf, lbuf):
        @plsc.parallel_loop(0, ROWS_PER_CHUNK * 100 // (COL_PERIOD * 16))
        def _(po):
            rbase = po * 4
            for vi in range(COL_PERIOD):
                ridx = rtab_v[pl.ds(vi * 16, 16)] + rbase
                cidx = ctab_v[pl.ds(vi * 16, 16)]
                x = plsc.load_gather(buf, [ridx, cidx])
                idx = bin_adj(x) * 100 + cidx
                plsc.addupdate_scatter(conf_v, [idx], x)

        @plsc.parallel_loop(0, ROWS_PER_CHUNK // 16, unroll=2)
        def _(li):
            lv = lbuf[pl.ds(li * 16, 16)]
            rl = iota + li * 16
            xg = plsc.load_gather(buf, [rl, lv])
            slot = ((bin_adj(xg) * 100 + lv) << 4) + iota
            plsc.addupdate_scatter(lacc_v, [slot], ones)

    @pl.when(wid < n_chunks)
    def _():
        start(wid, 0)

    @pl.when(wid + NW < n_chunks)
    def _():
        start(wid + NW, 1)

    n_outer = (n_chunks + 2 * NW - 1) // (2 * NW)

    def outer(i, carry):
        for b2 in range(2):
            c = wid + (2 * i + b2) * NW

            @pl.when(c < n_chunks)
            def _():
                wait(b2)
                process(bufs[b2], lbufs[b2])
                cn = c + 2 * NW

                @pl.when(cn < n_chunks)
                def _():
                    start(cn, b2)

        return carry

    lax.fori_loop(0, n_outer, outer, 0)

    @plsc.parallel_loop(0, ACC // 16)
    def _(ov):
        s = conf_v[pl.ds(ov * 16, 16)]
        gbase = iota16 + ov * 256
        for l in range(16):
            s = s - plsc.load_gather(lacc_v, [gbase + l])
        conf_v[pl.ds(ov * 16, 16)] = s

    pltpu.sync_copy(conf_v, out_hbm.at[wid])


def _final_body(inv_ncl, p_ref, o_ref):
    p = p_ref[...]
    s = jnp.sum(p, axis=0, keepdims=True)
    j = lax.broadcasted_iota(jnp.int32, (1, ACC), 1)
    m = (j >= 100) & (j < 1600)
    d = jnp.where(m, jnp.abs(s), 0.0)
    o_ref[...] = (jnp.sum(d) * inv_ncl).reshape(1, 1)


@jax.jit
def kernel(softmaxes, labels):
    n, num_classes = softmaxes.shape
    assert num_classes == 100 and n % ROWS_PER_CHUNK == 0
    n_chunks = n // ROWS_PER_CHUNK

    lab = labels.astype(jnp.int32)
    btab = jnp.repeat(jnp.linspace(0.0, 1.0, N_BINS + 1).astype(jnp.float32), 16)
    rowtab = jnp.arange(COL_PERIOD * 16, dtype=jnp.int32) // 100
    coltab = jnp.arange(COL_PERIOD * 16, dtype=jnp.int32) % 100

    mesh = plsc.VectorSubcoreMesh(core_axis_name="c", subcore_axis_name="s")
    sc = pl.kernel(
        functools.partial(_sc_body, n_chunks),
        out_type=jax.ShapeDtypeStruct((NW, ACC), jnp.float32),
        mesh=mesh,
        compiler_params=pltpu.CompilerParams(needs_layout_passes=False),
        scratch_types=[
            pltpu.VMEM((ROWS_PER_CHUNK, 100), jnp.float32),
            pltpu.VMEM((ROWS_PER_CHUNK, 100), jnp.float32),
            pltpu.VMEM((ROWS_PER_CHUNK,), jnp.int32),
            pltpu.VMEM((ROWS_PER_CHUNK,), jnp.int32),
            pltpu.VMEM((256,), jnp.float32),
            pltpu.VMEM((COL_PERIOD * 16,), jnp.int32),
            pltpu.VMEM((COL_PERIOD * 16,), jnp.int32),
            pltpu.VMEM((ACC,), jnp.float32),
            pltpu.VMEM((ACC * 16,), jnp.float32),
            pltpu.SemaphoreType.DMA,
            pltpu.SemaphoreType.DMA,
            pltpu.SemaphoreType.DMA,
            pltpu.SemaphoreType.DMA,
        ],
    )
    part = sc(softmaxes, lab, btab, rowtab, coltab)

    inv_ncl = 1.0 / (float(n) * float(num_classes))
    out = pl.pallas_call(
        functools.partial(_final_body, inv_ncl),
        out_shape=jax.ShapeDtypeStruct((1, 1), jnp.float32),
    )(part)
    return out[0, 0]


# speedup vs baseline: 4.9022x; 1.4927x over previous
import functools

import jax
import jax.numpy as jnp
from jax import lax
from jax.experimental import pallas as pl
from jax.experimental.pallas import tpu as pltpu
from jax.experimental.pallas import tpu_sc as plsc

N_BINS = 15
ACC = 1792
NW = 32
ROWS_PER_CHUNK = 160


def _sc_body(n_chunks, sm_hbm, lab_hbm, btab_hbm, out_hbm,
             buf0, buf1, lbuf0, lbuf1, btab_v, conf_v, lacc_v,
             s0, s1, sl0, sl1):
    cid = lax.axis_index("c")
    sid = lax.axis_index("s")
    wid = sid * 2 + cid

    pltpu.sync_copy(btab_hbm, btab_v)

    zf = jnp.zeros((16,), jnp.float32)

    @plsc.parallel_loop(0, ACC // 16, unroll=8)
    def _(i):
        conf_v[pl.ds(i * 16, 16)] = zf

    @plsc.parallel_loop(0, ACC, unroll=8)
    def _(i):
        lacc_v[pl.ds(i * 16, 16)] = zf

    iota = lax.iota(jnp.int32, 16)
    iota16 = iota * 16
    ones = jnp.ones((16,), jnp.float32)
    tail_mask = iota >= 12
    cvecs = [iota + 16 * vi for vi in range(6)]
    cvec_tail = iota + 84

    bufs = (buf0, buf1)
    lbufs = (lbuf0, lbuf1)
    sems = (s0, s1)
    lsems = (sl0, sl1)

    def start(c, b):
        off = pl.multiple_of(c * ROWS_PER_CHUNK, 8)
        pltpu.async_copy(sm_hbm.at[pl.ds(off, ROWS_PER_CHUNK)], bufs[b], sems[b])
        pltpu.async_copy(lab_hbm.at[pl.ds(off, ROWS_PER_CHUNK)], lbufs[b], lsems[b])

    def wait(b):
        pltpu.make_async_copy(
            sm_hbm.at[pl.ds(0, ROWS_PER_CHUNK)], bufs[b], sems[b]).wait()
        pltpu.make_async_copy(
            lab_hbm.at[pl.ds(0, ROWS_PER_CHUNK)], lbufs[b], lsems[b]).wait()

    def kof(x):
        return (x * 15.0 + 0.5).astype(jnp.int32)

    def badj_of(x, k):
        u = plsc.load_gather(btab_v, [(k << 4) + iota])
        return k + jnp.where(u < x, 1, 0)

    def process(buf, lbuf):
        @plsc.parallel_loop(0, ROWS_PER_CHUNK)
        def _(r):
            for vi in range(6):
                x = buf[r, pl.ds(vi * 16, 16)]
                idx = badj_of(x, kof(x)) * 100 + cvecs[vi]
                plsc.addupdate_scatter(conf_v, [idx], x)
            x = buf[r, pl.ds(84, 16)]
            idx = badj_of(x, kof(x)) * 100 + cvec_tail
            plsc.addupdate_scatter(conf_v, [idx], x, mask=tail_mask)

        @plsc.parallel_loop(0, ROWS_PER_CHUNK // 16, unroll=2)
        def _(li):
            lv = lbuf[pl.ds(li * 16, 16)]
            rl = iota + li * 16
            xg = plsc.load_gather(buf, [rl, lv])
            slot = ((badj_of(xg, kof(xg)) * 100 + lv) << 4) + iota
            plsc.addupdate_scatter(lacc_v, [slot], ones)

    @pl.when(wid < n_chunks)
    def _():
        start(wid, 0)

    @pl.when(wid + NW < n_chunks)
    def _():
        start(wid + NW, 1)

    n_outer = (n_chunks + 2 * NW - 1) // (2 * NW)

    def outer(i, carry):
        for b2 in range(2):
            c = wid + (2 * i + b2) * NW

            @pl.when(c < n_chunks)
            def _():
                wait(b2)
                process(bufs[b2], lbufs[b2])
                cn = c + 2 * NW

                @pl.when(cn < n_chunks)
                def _():
                    start(cn, b2)

        return carry

    lax.fori_loop(0, n_outer, outer, 0)

    @plsc.parallel_loop(0, ACC // 16)
    def _(ov):
        s = conf_v[pl.ds(ov * 16, 16)]
        gbase = iota16 + ov * 256
        for l in range(16):
            s = s - plsc.load_gather(lacc_v, [gbase + l])
        conf_v[pl.ds(ov * 16, 16)] = s

    pltpu.sync_copy(conf_v, out_hbm.at[wid])


def _final_body(inv_ncl, p_ref, o_ref):
    p = p_ref[...]
    s = jnp.sum(p, axis=0, keepdims=True)
    j = lax.broadcasted_iota(jnp.int32, (1, ACC), 1)
    m = (j >= 100) & (j < 1600)
    d = jnp.where(m, jnp.abs(s), 0.0)
    o_ref[...] = (jnp.sum(d) * inv_ncl).reshape(1, 1)


@jax.jit
def kernel(softmaxes, labels):
    n, num_classes = softmaxes.shape
    assert num_classes == 100 and n % ROWS_PER_CHUNK == 0
    n_chunks = n // ROWS_PER_CHUNK

    lab = labels.astype(jnp.int32)
    btab = jnp.repeat(jnp.linspace(0.0, 1.0, N_BINS + 1).astype(jnp.float32), 16)

    mesh = plsc.VectorSubcoreMesh(core_axis_name="c", subcore_axis_name="s")
    sc = pl.kernel(
        functools.partial(_sc_body, n_chunks),
        out_type=jax.ShapeDtypeStruct((NW, ACC), jnp.float32),
        mesh=mesh,
        compiler_params=pltpu.CompilerParams(needs_layout_passes=False),
        scratch_types=[
            pltpu.VMEM((ROWS_PER_CHUNK, 100), jnp.float32),
            pltpu.VMEM((ROWS_PER_CHUNK, 100), jnp.float32),
            pltpu.VMEM((ROWS_PER_CHUNK,), jnp.int32),
            pltpu.VMEM((ROWS_PER_CHUNK,), jnp.int32),
            pltpu.VMEM((256,), jnp.float32),
            pltpu.VMEM((ACC,), jnp.float32),
            pltpu.VMEM((ACC * 16,), jnp.float32),
            pltpu.SemaphoreType.DMA,
            pltpu.SemaphoreType.DMA,
            pltpu.SemaphoreType.DMA,
            pltpu.SemaphoreType.DMA,
        ],
    )
    part = sc(softmaxes, lab, btab)

    inv_ncl = 1.0 / (float(n) * float(num_classes))
    out = pl.pallas_call(
        functools.partial(_final_body, inv_ncl),
        out_shape=jax.ShapeDtypeStruct((1, 1), jnp.float32),
    )(part)
    return out[0, 0]
